# baseline (speedup 1.0000x reference)
import functools

import jax
import jax.numpy as jnp
from jax import lax
from jax.experimental import pallas as pl
from jax.experimental.pallas import tpu as pltpu
from jax.experimental.pallas import tpu_sc as plsc

_OUT = 128
_MAX = 47
_SUM = 47
_RND = _OUT - 2 * _MAX
_S = 196
_POOL3 = _S - _MAX - _SUM

_B = 8
_I = 32
_O = 32
_H = 16
_PAIRS = _B * _I
_ROWS = _O * _H

_NC = 2
_NS = 16
_NW = _NC * _NS
_OCHUNK = 8
_SUB = _O // _OCHUNK
_TROWS = _OCHUNK * _H
_NTASK = _PAIRS * _SUB
_TPW = _NTASK // _NW


def _build_q():
    idx_lucky = jax.random.permutation(jax.random.key(42), _POOL3)[:_RND]
    idx43 = jax.random.permutation(jax.random.key(43), _OUT)
    inv43 = jnp.zeros(_OUT, jnp.int32).at[idx43].set(jnp.arange(_OUT, dtype=jnp.int32))
    invlucky = jnp.full(_POOL3, _OUT, jnp.int32).at[idx_lucky].set(
        jnp.arange(_RND, dtype=jnp.int32))
    kept = invlucky < _RND
    t3 = jnp.where(kept, inv43[jnp.clip(2 * _MAX + invlucky, 0, _OUT - 1)], 999)
    t = jnp.concatenate([inv43[: 2 * _MAX], t3])
    q = (t[:, None] == jnp.arange(_OUT, dtype=jnp.int32)[None, :]).astype(jnp.float32)
    return q


def _mm(a, b, precision=None):
    return jnp.dot(a, b, preferred_element_type=jnp.float32,
                   precision=precision)


def _idx_body(route_ref, q_ref, out_ref):
    r = route_ref[0]
    i0 = lax.broadcasted_iota(jnp.int32, (_S, _S), 0)
    i1 = lax.broadcasted_iota(jnp.int32, (_S, _S), 1)
    eye = (i0 == i1).astype(jnp.float32)
    ones_mat = jnp.ones((_S, _S), jnp.float32)

    def colb(v_row, precision=None):
        return _mm(v_row * eye, ones_mat, precision)

    x_row = jnp.max(r, axis=0, keepdims=True)
    y_row = jnp.sum(r, axis=0, keepdims=True)
    x_cb = colb(x_row, lax.Precision.HIGHEST)
    y_cb = colb(y_row, lax.Precision.HIGHEST)

    ones_row = jnp.ones((1, _S), jnp.float32)
    m1 = jnp.where((x_cb > x_row) | ((x_cb == x_row) & (i0 < i1)), 1.0, 0.0)
    r1_row = _mm(ones_row, m1)
    r1_cb = colb(r1_row)

    pool_cb = r1_cb >= _MAX
    m2 = jnp.where(
        pool_cb & ((y_cb > y_row) | ((y_cb == y_row) & (r1_cb < r1_row))),
        1.0, 0.0)
    r2_row = _mm(ones_row, m2)

    c_row = jnp.where(r1_row < _MAX, r1_row, _MAX + r2_row)
    c_cb = colb(c_row)
    cmat = (c_cb == i1.astype(jnp.float32)).astype(jnp.float32)

    iota_row = lax.broadcasted_iota(jnp.int32, (1, _S), 1).astype(jnp.float32)
    invc = _mm(iota_row, cmat)
    src_row = _mm(invc, q_ref[...])
    out_ref[0] = src_row.astype(jnp.int32)


def _compute_src(route, q):
    route_p = route.reshape(_PAIRS, _O, _S)
    src = pl.pallas_call(
        _idx_body,
        grid=(_PAIRS,),
        in_specs=[
            pl.BlockSpec((1, _O, _S), lambda p: (p, 0, 0)),
            pl.BlockSpec((_S, _OUT), lambda p: (0, 0)),
        ],
        out_specs=pl.BlockSpec((1, 1, _OUT), lambda p: (p, 0, 0)),
        out_shape=jax.ShapeDtypeStruct((_PAIRS, 1, _OUT), jnp.int32),
    )(route_p, q)
    return src.reshape(_PAIRS * _OUT)


def _sc_gather_body(votes_hbm, src_hbm, out_hbm, in_v, out_v, src_v):
    cid = lax.axis_index("c")
    sid = lax.axis_index("s")
    wid = sid * _NC + cid

    def task(t, carry):
        task_id = wid * _TPW + t
        pair = task_id // _SUB
        sub = task_id % _SUB
        in_base = pair * (_ROWS * _S) + sub * (_TROWS * _S)
        out_base = pair * (_ROWS * _OUT) + sub * (_TROWS * _OUT)
        pltpu.sync_copy(src_hbm.at[pl.ds(pair * _OUT, _OUT)], src_v)
        pltpu.sync_copy(votes_hbm.at[pl.ds(in_base, _TROWS * _S)], in_v)

        def row(r, c2):
            rb = r * _S
            ob = r * _OUT
            for kc in range(_OUT // 16):
                idx = src_v[pl.ds(kc * 16, 16)] + rb
                out_v[pl.ds(ob + kc * 16, 16)] = plsc.load_gather(in_v, [idx])
            return c2

        lax.fori_loop(0, _TROWS, row, 0)
        pltpu.sync_copy(out_v, out_hbm.at[pl.ds(out_base, _TROWS * _OUT)])
        return carry

    lax.fori_loop(0, _TPW, task, 0)


@functools.cache
def _make_sc_gather():
    mesh = plsc.VectorSubcoreMesh(
        core_axis_name="c", subcore_axis_name="s",
        num_cores=_NC, num_subcores=_NS)
    return pl.kernel(
        _sc_gather_body,
        out_type=jax.ShapeDtypeStruct((_B * _I * _O * _H * _OUT,), jnp.float32),
        mesh=mesh,
        scratch_types=[
            pltpu.VMEM((_TROWS * _S,), jnp.float32),
            pltpu.VMEM((_TROWS * _OUT,), jnp.float32),
            pltpu.VMEM((_OUT,), jnp.int32),
        ],
        compiler_params=pltpu.CompilerParams(needs_layout_passes=False),
    )


def _sc_gather(votes_flat, src_flat):
    return _make_sc_gather()(votes_flat, src_flat)


def kernel(votes, route):
    b, input_dim, output_dim, h = votes.shape[:4]
    votes = votes.reshape(b, input_dim, output_dim, h, -1)
    route = route.reshape(b, input_dim, output_dim, -1)
    q = _build_q()

    src = _compute_src(route, q)
    out_flat = _sc_gather(votes.reshape(-1), src)
    return out_flat.reshape(b, input_dim, output_dim, h, _OUT)[..., None]


# speedup vs baseline: 2.3774x; 2.3774x over previous
import jax
import jax.numpy as jnp
from jax import lax
from jax.experimental import pallas as pl

_OUT = 128
_MAX = 47
_SUM = 47
_RND = _OUT - 2 * _MAX
_S = 196
_POOL3 = _S - _MAX - _SUM


def _build_q():
    idx_lucky = jax.random.permutation(jax.random.key(42), _POOL3)[:_RND]
    idx43 = jax.random.permutation(jax.random.key(43), _OUT)
    inv43 = jnp.zeros(_OUT, jnp.int32).at[idx43].set(jnp.arange(_OUT, dtype=jnp.int32))
    invlucky = jnp.full(_POOL3, _OUT, jnp.int32).at[idx_lucky].set(
        jnp.arange(_RND, dtype=jnp.int32))
    kept = invlucky < _RND
    t3 = jnp.where(kept, inv43[jnp.clip(2 * _MAX + invlucky, 0, _OUT - 1)], 999)
    t = jnp.concatenate([inv43[: 2 * _MAX], t3])
    q = (t[:, None] == jnp.arange(_OUT, dtype=jnp.int32)[None, :]).astype(jnp.float32)
    return q


def _mm(a, b, precision=None):
    return jnp.dot(a, b, preferred_element_type=jnp.float32,
                   precision=precision)


def _body(route_ref, votes_ref, q_ref, out_ref):
    for ii in range(route_ref.shape[1]):
        _one_pair(route_ref, votes_ref, q_ref, out_ref, ii)


def _one_pair(route_ref, votes_ref, q_ref, out_ref, ii):
    r = route_ref[0, ii]
    i0 = lax.broadcasted_iota(jnp.int32, (_S, _S), 0)
    i1 = lax.broadcasted_iota(jnp.int32, (_S, _S), 1)
    eye = (i0 == i1).astype(jnp.float32)
    ones_row = jnp.ones((1, _S), jnp.float32)
    ones_mat = jnp.ones((_S, _S), jnp.float32)

    def colb(v_row, precision=None):
        return _mm(v_row * eye, ones_mat, precision)

    x_row = jnp.max(r, axis=0, keepdims=True)
    y_row = jnp.sum(r, axis=0, keepdims=True)
    x_cb = colb(x_row, lax.Precision.HIGHEST)
    y_cb = colb(y_row, lax.Precision.HIGHEST)

    m1 = jnp.where((x_cb > x_row) | ((x_cb == x_row) & (i0 < i1)), 1.0, 0.0)
    r1_row = _mm(ones_row, m1)
    r1_cb = colb(r1_row)

    pool_cb = r1_cb >= _MAX
    m2 = jnp.where(
        pool_cb & ((y_cb > y_row) | ((y_cb == y_row) & (r1_cb < r1_row))),
        1.0, 0.0)
    r2_row = _mm(ones_row, m2)

    c_row = jnp.where(r1_row < _MAX, r1_row, _MAX + r2_row)
    c_cb = colb(c_row)
    cmat = (c_cb == i1.astype(jnp.float32)).astype(jnp.float32)
    p = _mm(cmat, q_ref[...]).astype(jnp.bfloat16)

    v = votes_ref[0, ii]
    v1 = v.astype(jnp.bfloat16)
    rem = v - v1.astype(jnp.float32)
    v2 = rem.astype(jnp.bfloat16)
    v3 = (rem - v2.astype(jnp.float32)).astype(jnp.bfloat16)
    dn = (((0,), (0,)), ((), ()))
    acc = (lax.dot_general(v1, p, dn, preferred_element_type=jnp.float32)
           + lax.dot_general(v2, p, dn, preferred_element_type=jnp.float32)
           + lax.dot_general(v3, p, dn, preferred_element_type=jnp.float32))
    out_ref[0, ii] = acc.reshape(16, 32, _OUT).transpose(1, 0, 2)


def kernel(votes, route):
    b, input_dim, output_dim, h = votes.shape[:4]
    votes_nat = votes.transpose(0, 1, 4, 5, 3, 2).reshape(
        b, input_dim, _S, h * output_dim)
    route = route.reshape(b, input_dim, output_dim, -1)
    q = _build_q()

    pb = 4
    out = pl.pallas_call(
        _body,
        grid=(b, input_dim // pb),
        in_specs=[
            pl.BlockSpec((1, pb, output_dim, _S), lambda bi, ii: (bi, ii, 0, 0)),
            pl.BlockSpec((1, pb, _S, h * output_dim),
                         lambda bi, ii: (bi, ii, 0, 0)),
            pl.BlockSpec((_S, _OUT), lambda bi, ii: (0, 0)),
        ],
        out_specs=pl.BlockSpec((1, pb, output_dim, h, _OUT),
                               lambda bi, ii: (bi, ii, 0, 0, 0)),
        out_shape=jax.ShapeDtypeStruct((b, input_dim, output_dim, h, _OUT),
                                       jnp.float32),
    )(route, votes_nat, q)
    return out[..., None]
